# Initial kernel scaffold; baseline (speedup 1.0000x reference)
import jax
import jax.numpy as jnp
from jax.experimental import pallas as pl


def kernel(xs, edge_indices, edge_attrs, W_pre, b_pre, W_e1, b_e1, W_e2, b_e2, l1_Wmsg, l1_bmsg, l1_Wih, l1_Whh, l1_bih, l1_bhh, l2_Wmsg, l2_bmsg, l2_Wih, l2_Whh, l2_bih, l2_bhh, g_Wih, g_Whh, g_bih, g_bhh, W_post, b_post, W_reg, b_reg):
    raise NotImplementedError("write your pallas kernel here")


# speedup vs baseline: 2.5607x; 2.5607x over previous
import functools

import jax
import jax.numpy as jnp
from jax import lax
from jax.experimental import pallas as pl
from jax.experimental.pallas import tpu as pltpu
from jax.experimental.pallas import tpu_sc as plsc

T, N, E = 4, 10000, 320000
D, H, ED = 128, 128, 16
NP = 10240
RB = 1024
EB = 2048

_NC, _NS = 2, 16
NW = _NC * _NS
CHUNK = 128
NCHUNKS = E // CHUNK
BASE_CH = NCHUNKS // NW
EXTRA_CH = NCHUNKS - BASE_CH * NW
RPT = NP // _NS

_F32 = jnp.float32


def _pre_body(x_ref, wp_ref, bp_ref, w1_ref, b1_ref, xt_ref, u1_ref):
    xt = jnp.maximum(
        jnp.dot(x_ref[...], wp_ref[...], preferred_element_type=_F32)
        + bp_ref[...], 0.0)
    xt_ref[...] = xt
    u1_ref[...] = jnp.maximum(
        jnp.dot(xt, w1_ref[...], preferred_element_type=_F32)
        + b1_ref[...], 0.0)


def _pre_stage(x_flat, wpT, bp, w1T, b1):
    nblk = x_flat.shape[0] // RB
    return pl.pallas_call(
        _pre_body,
        grid=(nblk,),
        in_specs=[
            pl.BlockSpec((RB, D), lambda i: (i, 0)),
            pl.BlockSpec((D, H), lambda i: (0, 0)),
            pl.BlockSpec((1, H), lambda i: (0, 0)),
            pl.BlockSpec((H, H), lambda i: (0, 0)),
            pl.BlockSpec((1, H), lambda i: (0, 0)),
        ],
        out_specs=[
            pl.BlockSpec((RB, H), lambda i: (i, 0)),
            pl.BlockSpec((RB, H), lambda i: (i, 0)),
        ],
        out_shape=[
            jax.ShapeDtypeStruct((x_flat.shape[0], H), _F32),
            jax.ShapeDtypeStruct((x_flat.shape[0], H), _F32),
        ],
    )(x_flat, wpT, bp, w1T, b1)


def _ew_body(ea_ref, w1_ref, b1_ref, w2_ref, b2_ref, out_ref):
    h = jnp.maximum(
        jnp.dot(ea_ref[...], w1_ref[...], preferred_element_type=_F32)
        + b1_ref[...], 0.0)
    v = jnp.sum(h * w2_ref[...], axis=1) + b2_ref[0, 0]
    out_ref[...] = jax.nn.sigmoid(v).reshape(out_ref.shape)


def _ew_stage(ea_flat, w1T, b1, w2, b2):
    rows = ea_flat.shape[0]
    nblk = rows // EB
    out = pl.pallas_call(
        _ew_body,
        grid=(nblk,),
        in_specs=[
            pl.BlockSpec((EB, ED), lambda i: (i, 0)),
            pl.BlockSpec((ED, H // 2), lambda i: (0, 0)),
            pl.BlockSpec((1, H // 2), lambda i: (0, 0)),
            pl.BlockSpec((1, H // 2), lambda i: (0, 0)),
            pl.BlockSpec((1, 1), lambda i: (0, 0)),
        ],
        out_specs=pl.BlockSpec((EB // 128, 128), lambda i: (i, 0)),
        out_shape=jax.ShapeDtypeStruct((rows // 128, 128), _F32),
    )(ea_flat, w1T, b1, w2, b2)
    return out.reshape(rows)


def _make_mpgru_body(with_next):
    def body(p_ref, dg_ref, x_ref, wih_ref, whh_ref, bih_ref, bhh_ref,
             *rest):
        if with_next:
            wn_ref, bn_ref, h_ref, un_ref = rest
        else:
            (h_ref,) = rest
        deg = jnp.clip(dg_ref[0] + dg_ref[1], 1.0, None)
        agg = (p_ref[0] + p_ref[1]) / deg
        x = x_ref[...]
        gi = jnp.dot(agg, wih_ref[...], preferred_element_type=_F32) \
            + bih_ref[...]
        gh = jnp.dot(x, whh_ref[...], preferred_element_type=_F32) \
            + bhh_ref[...]
        r = jax.nn.sigmoid(gi[:, :H] + gh[:, :H])
        z = jax.nn.sigmoid(gi[:, H:2 * H] + gh[:, H:2 * H])
        n = jnp.tanh(gi[:, 2 * H:] + r * gh[:, 2 * H:])
        out = (1.0 - z) * n + z * x
        nrm = jnp.maximum(
            jnp.sqrt(jnp.sum(out * out, axis=1, keepdims=True)), 1e-12)
        out = out / nrm
        h_ref[...] = out
        if with_next:
            un_ref[...] = jnp.maximum(
                jnp.dot(out, wn_ref[...], preferred_element_type=_F32)
                + bn_ref[...], 0.0)
    return body


def _mpgru_stage(parts, degs, x, wihT, whhT, bih, bhh, wnT=None, bn=None):
    with_next = wnT is not None
    nblk = NP // RB
    in_specs = [
        pl.BlockSpec((2, RB, H), lambda i: (0, i, 0)),
        pl.BlockSpec((2, RB, 1), lambda i: (0, i, 0)),
        pl.BlockSpec((RB, H), lambda i: (i, 0)),
        pl.BlockSpec((H, 3 * H), lambda i: (0, 0)),
        pl.BlockSpec((H, 3 * H), lambda i: (0, 0)),
        pl.BlockSpec((1, 3 * H), lambda i: (0, 0)),
        pl.BlockSpec((1, 3 * H), lambda i: (0, 0)),
    ]
    args = [parts, degs, x, wihT, whhT, bih, bhh]
    out_specs = [pl.BlockSpec((RB, H), lambda i: (i, 0))]
    out_shape = [jax.ShapeDtypeStruct((NP, H), _F32)]
    if with_next:
        in_specs += [
            pl.BlockSpec((H, H), lambda i: (0, 0)),
            pl.BlockSpec((1, H), lambda i: (0, 0)),
        ]
        args += [wnT, bn]
        out_specs.append(pl.BlockSpec((RB, H), lambda i: (i, 0)))
        out_shape.append(jax.ShapeDtypeStruct((NP, H), _F32))
    res = pl.pallas_call(
        _make_mpgru_body(with_next),
        grid=(nblk,),
        in_specs=in_specs,
        out_specs=out_specs,
        out_shape=out_shape,
    )(*args)
    return res if with_next else (res[0], None)


def _temporal_body(hs_ref, wih_ref, whh_ref, bih_ref, bhh_ref,
                   wpo_ref, bpo_ref, wrg_ref, brg_ref, y_ref):
    ht = jnp.zeros((RB, H), _F32)
    for t in range(T):
        xt = hs_ref[t]
        gi = jnp.dot(xt, wih_ref[...], preferred_element_type=_F32) \
            + bih_ref[...]
        gh = jnp.dot(ht, whh_ref[...], preferred_element_type=_F32) \
            + bhh_ref[...]
        r = jax.nn.sigmoid(gi[:, :H] + gh[:, :H])
        z = jax.nn.sigmoid(gi[:, H:2 * H] + gh[:, H:2 * H])
        n = jnp.tanh(gi[:, 2 * H:] + r * gh[:, 2 * H:])
        ht = (1.0 - z) * n + z * ht
    hp = jnp.maximum(
        jnp.dot(ht, wpo_ref[...], preferred_element_type=_F32)
        + bpo_ref[...], 0.0)
    y_ref[...] = jnp.dot(hp, wrg_ref[...], preferred_element_type=_F32) \
        + brg_ref[...]


def _temporal_stage(h_seq, wihT, whhT, bih, bhh, wpoT, bpo, wrgT, brg):
    nblk = NP // RB
    return pl.pallas_call(
        _temporal_body,
        grid=(nblk,),
        in_specs=[
            pl.BlockSpec((T, RB, H), lambda i: (0, i, 0)),
            pl.BlockSpec((H, 3 * H), lambda i: (0, 0)),
            pl.BlockSpec((H, 3 * H), lambda i: (0, 0)),
            pl.BlockSpec((1, 3 * H), lambda i: (0, 0)),
            pl.BlockSpec((1, 3 * H), lambda i: (0, 0)),
            pl.BlockSpec((H, H), lambda i: (0, 0)),
            pl.BlockSpec((1, H), lambda i: (0, 0)),
            pl.BlockSpec((H, 128), lambda i: (0, 0)),
            pl.BlockSpec((1, 128), lambda i: (0, 0)),
        ],
        out_specs=pl.BlockSpec((RB, 128), lambda i: (i, 0)),
        out_shape=jax.ShapeDtypeStruct((NP, 128), _F32),
    )(h_seq, wihT, whhT, bih, bhh, wpoT, bpo, wrgT, brg)


def _make_edge_pass(with_deg):
    mesh = plsc.VectorSubcoreMesh(core_axis_name="c", subcore_axis_name="s")
    out_type = [jax.ShapeDtypeStruct((2, NP, H), _F32)]
    if with_deg:
        out_type.append(jax.ShapeDtypeStruct((2, NP), _F32))
    scratch = [
        pltpu.VMEM((CHUNK,), jnp.int32),
        pltpu.VMEM((CHUNK,), jnp.int32),
        pltpu.VMEM((CHUNK,), _F32),
        pltpu.VMEM((CHUNK, H), _F32),
        pltpu.VMEM((CHUNK, H), _F32),
        pltpu.VMEM_SHARED((NP, H), _F32),
    ]
    if with_deg:
        scratch += [
            pltpu.VMEM((CHUNK,), _F32),
            pltpu.VMEM((RPT,), _F32),
            pltpu.VMEM_SHARED((NP,), _F32),
        ]
    scratch.append(pltpu.SemaphoreType.DMA)

    def body(u_hbm, src_hbm, dst_hbm, ew_hbm, *rest):
        if with_deg:
            (acc_out, deg_out, src_v, dst_v, ew_v, rows_v, zrows_v,
             acc_s, ones_v, zdeg_v, deg_s, sem) = rest
        else:
            (acc_out, src_v, dst_v, ew_v, rows_v, zrows_v,
             acc_s, sem) = rest
        cid = lax.axis_index("c")
        sid = lax.axis_index("s")
        wid = sid * _NC + cid

        def zrow(i, c):
            for j in range(H // 16):
                zrows_v[i, pl.ds(j * 16, 16)] = jnp.zeros((16,), _F32)
            return c
        lax.fori_loop(0, CHUNK, zrow, 0)
        if with_deg:
            def zfill(i, c):
                ones_v[pl.ds(i * 16, 16)] = jnp.ones((16,), _F32)
                return c
            lax.fori_loop(0, CHUNK // 16, zfill, 0)

            def zdeg(i, c):
                zdeg_v[pl.ds(i * 16, 16)] = jnp.zeros((16,), _F32)
                return c
            lax.fori_loop(0, RPT // 16, zdeg, 0)

        base_r = sid * RPT
        for b in range(RPT // CHUNK):
            pltpu.sync_copy(zrows_v,
                            acc_s.at[pl.ds(base_r + b * CHUNK, CHUNK)])
        if with_deg:
            pltpu.sync_copy(zdeg_v, deg_s.at[pl.ds(base_r, RPT)])
        plsc.subcore_barrier()

        nch = BASE_CH + jnp.where(wid < EXTRA_CH, 1, 0)

        def chunk_body(k, c):
            base = (wid + NW * k) * CHUNK
            pltpu.sync_copy(src_hbm.at[pl.ds(base, CHUNK)], src_v)
            pltpu.sync_copy(dst_hbm.at[pl.ds(base, CHUNK)], dst_v)
            pltpu.sync_copy(ew_hbm.at[pl.ds(base, CHUNK)], ew_v)
            pltpu.async_copy(u_hbm.at[src_v], rows_v, sem).wait()

            def gbody(g, cc):
                ewg = ew_v[pl.ds(g * 16, 16)]
                e0 = g * 16
                for k in range(16):
                    s = ewg[k]
                    for j in range(H // 16):
                        sl = pl.ds(j * 16, 16)
                        rows_v[e0 + k, sl] = rows_v[e0 + k, sl] * s
                return cc
            lax.fori_loop(0, CHUNK // 16, gbody, 0)

            pltpu.sync_copy(rows_v, acc_s.at[dst_v], add=True)
            if with_deg:
                pltpu.sync_copy(ones_v, deg_s.at[dst_v], add=True)
            return c
        lax.fori_loop(0, nch, chunk_body, 0)
        plsc.subcore_barrier()

        pltpu.sync_copy(acc_s.at[pl.ds(base_r, RPT)],
                        acc_out.at[cid, pl.ds(base_r, RPT)])
        if with_deg:
            pltpu.sync_copy(deg_s.at[pl.ds(base_r, RPT)],
                            deg_out.at[cid, pl.ds(base_r, RPT)])

    return functools.partial(pl.kernel, mesh=mesh, out_type=out_type,
                             scratch_types=scratch)(body)


@functools.lru_cache(maxsize=None)
def _get_edge_pass(with_deg):
    return _make_edge_pass(with_deg)


def _edge_pass(u, src, dst, ew, with_deg):
    if with_deg:
        return _get_edge_pass(True)(u, src, dst, ew)
    (acc,) = _get_edge_pass(False)(u, src, dst, ew)
    return acc, None


def kernel(xs, edge_indices, edge_attrs, W_pre, b_pre, W_e1, b_e1, W_e2,
           b_e2, l1_Wmsg, l1_bmsg, l1_Wih, l1_Whh, l1_bih, l1_bhh,
           l2_Wmsg, l2_bmsg, l2_Wih, l2_Whh, l2_bih, l2_bhh,
           g_Wih, g_Whh, g_bih, g_bhh, W_post, b_post, W_reg, b_reg):
    xs_p = jnp.pad(xs, ((0, 0), (0, NP - N), (0, 0)))
    x_flat = xs_p.reshape(T * NP, D)

    xt_all, u1_all = _pre_stage(
        x_flat, W_pre.T, b_pre[None], l1_Wmsg.T, l1_bmsg[None])
    xt_all = xt_all.reshape(T, NP, H)
    u1_all = u1_all.reshape(T, NP, H)

    ew_all = _ew_stage(
        edge_attrs.reshape(T * E, ED), W_e1.T, b_e1[None], W_e2,
        b_e2.reshape(1, 1)).reshape(T, E)

    l1_wihT, l1_whhT = l1_Wih.T, l1_Whh.T
    l2_wihT, l2_whhT = l2_Wih.T, l2_Whh.T

    outs = []
    for t in range(T):
        src = edge_indices[t, 0]
        dst = edge_indices[t, 1]
        ew = ew_all[t]
        x = xt_all[t]
        u = u1_all[t]

        acc, deg = _edge_pass(u, src, dst, ew, True)
        degs = deg.reshape(2, NP, 1)
        x, u2 = _mpgru_stage(acc, degs, x, l1_wihT, l1_whhT,
                             l1_bih[None], l1_bhh[None],
                             l2_Wmsg.T, l2_bmsg[None])
        acc2, _ = _edge_pass(u2, src, dst, ew, False)
        x, _ = _mpgru_stage(acc2, degs, x, l2_wihT, l2_whhT,
                            l2_bih[None], l2_bhh[None])
        outs.append(x)

    h_seq = jnp.stack(outs, axis=0)
    wrgT = jnp.pad(W_reg.T, ((0, 0), (0, 128 - 3)))
    brg = jnp.pad(b_reg, (0, 128 - 3))[None]
    y = _temporal_stage(h_seq, g_Wih.T, g_Whh.T, g_bih[None], g_bhh[None],
                        W_post.T, b_post[None], wrgT, brg)
    return y[:N, :3]


# speedup vs baseline: 4.6203x; 1.8043x over previous
import functools

import jax
import jax.numpy as jnp
from jax import lax
from jax.experimental import pallas as pl
from jax.experimental.pallas import tpu as pltpu
from jax.experimental.pallas import tpu_sc as plsc

T, N, E = 4, 10000, 320000
D, H, ED = 128, 128, 16
NP = 10240
RB = 1024
EB = 2048

_NC, _NS = 2, 16
NW = _NC * _NS
CHUNK = 64
NCHUNKS = E // CHUNK
BASE_CH = NCHUNKS // NW
EXTRA_CH = NCHUNKS - BASE_CH * NW
RPT = NP // _NS

_F32 = jnp.float32


def _pre_body(x_ref, wp_ref, bp_ref, w1_ref, b1_ref, xt_ref, u1_ref):
    xt = jnp.maximum(
        jnp.dot(x_ref[...], wp_ref[...], preferred_element_type=_F32)
        + bp_ref[...], 0.0)
    xt_ref[...] = xt
    u1_ref[...] = jnp.maximum(
        jnp.dot(xt, w1_ref[...], preferred_element_type=_F32)
        + b1_ref[...], 0.0)


def _pre_stage(x_flat, wpT, bp, w1T, b1):
    nblk = x_flat.shape[0] // RB
    return pl.pallas_call(
        _pre_body,
        grid=(nblk,),
        in_specs=[
            pl.BlockSpec((RB, D), lambda i: (i, 0)),
            pl.BlockSpec((D, H), lambda i: (0, 0)),
            pl.BlockSpec((1, H), lambda i: (0, 0)),
            pl.BlockSpec((H, H), lambda i: (0, 0)),
            pl.BlockSpec((1, H), lambda i: (0, 0)),
        ],
        out_specs=[
            pl.BlockSpec((RB, H), lambda i: (i, 0)),
            pl.BlockSpec((RB, H), lambda i: (i, 0)),
        ],
        out_shape=[
            jax.ShapeDtypeStruct((x_flat.shape[0], H), _F32),
            jax.ShapeDtypeStruct((x_flat.shape[0], H), _F32),
        ],
    )(x_flat, wpT, bp, w1T, b1)


def _ew_body(ea_ref, w1_ref, b1_ref, w2_ref, b2_ref, out_ref):
    h = jnp.maximum(
        jnp.dot(ea_ref[...], w1_ref[...], preferred_element_type=_F32)
        + b1_ref[...], 0.0)
    v = jnp.sum(h * w2_ref[...], axis=1) + b2_ref[0, 0]
    out_ref[...] = jax.nn.sigmoid(v).reshape(out_ref.shape)


def _ew_stage(ea_flat, w1T, b1, w2, b2):
    rows = ea_flat.shape[0]
    nblk = rows // EB
    out = pl.pallas_call(
        _ew_body,
        grid=(nblk,),
        in_specs=[
            pl.BlockSpec((EB, ED), lambda i: (i, 0)),
            pl.BlockSpec((ED, H // 2), lambda i: (0, 0)),
            pl.BlockSpec((1, H // 2), lambda i: (0, 0)),
            pl.BlockSpec((1, H // 2), lambda i: (0, 0)),
            pl.BlockSpec((1, 1), lambda i: (0, 0)),
        ],
        out_specs=pl.BlockSpec((EB // 128, 128), lambda i: (i, 0)),
        out_shape=jax.ShapeDtypeStruct((rows // 128, 128), _F32),
    )(ea_flat, w1T, b1, w2, b2)
    return out.reshape(rows)


def _make_mpgru_body(with_next):
    def body(p_ref, dg_ref, x_ref, wih_ref, whh_ref, bih_ref, bhh_ref,
             *rest):
        if with_next:
            wn_ref, bn_ref, h_ref, un_ref = rest
        else:
            (h_ref,) = rest
        deg = jnp.clip(dg_ref[0] + dg_ref[1], 1.0, None)
        agg = (p_ref[0] + p_ref[1]) / deg
        x = x_ref[...]
        gi = jnp.dot(agg, wih_ref[...], preferred_element_type=_F32) \
            + bih_ref[...]
        gh = jnp.dot(x, whh_ref[...], preferred_element_type=_F32) \
            + bhh_ref[...]
        r = jax.nn.sigmoid(gi[:, :H] + gh[:, :H])
        z = jax.nn.sigmoid(gi[:, H:2 * H] + gh[:, H:2 * H])
        n = jnp.tanh(gi[:, 2 * H:] + r * gh[:, 2 * H:])
        out = (1.0 - z) * n + z * x
        nrm = jnp.maximum(
            jnp.sqrt(jnp.sum(out * out, axis=1, keepdims=True)), 1e-12)
        out = out / nrm
        h_ref[...] = out
        if with_next:
            un_ref[...] = jnp.maximum(
                jnp.dot(out, wn_ref[...], preferred_element_type=_F32)
                + bn_ref[...], 0.0)
    return body


def _mpgru_stage(parts, degs, x, wihT, whhT, bih, bhh, wnT=None, bn=None):
    with_next = wnT is not None
    nblk = NP // RB
    in_specs = [
        pl.BlockSpec((2, RB, H), lambda i: (0, i, 0)),
        pl.BlockSpec((2, RB, 1), lambda i: (0, i, 0)),
        pl.BlockSpec((RB, H), lambda i: (i, 0)),
        pl.BlockSpec((H, 3 * H), lambda i: (0, 0)),
        pl.BlockSpec((H, 3 * H), lambda i: (0, 0)),
        pl.BlockSpec((1, 3 * H), lambda i: (0, 0)),
        pl.BlockSpec((1, 3 * H), lambda i: (0, 0)),
    ]
    args = [parts, degs, x, wihT, whhT, bih, bhh]
    out_specs = [pl.BlockSpec((RB, H), lambda i: (i, 0))]
    out_shape = [jax.ShapeDtypeStruct((NP, H), _F32)]
    if with_next:
        in_specs += [
            pl.BlockSpec((H, H), lambda i: (0, 0)),
            pl.BlockSpec((1, H), lambda i: (0, 0)),
        ]
        args += [wnT, bn]
        out_specs.append(pl.BlockSpec((RB, H), lambda i: (i, 0)))
        out_shape.append(jax.ShapeDtypeStruct((NP, H), _F32))
    res = pl.pallas_call(
        _make_mpgru_body(with_next),
        grid=(nblk,),
        in_specs=in_specs,
        out_specs=out_specs,
        out_shape=out_shape,
    )(*args)
    return res if with_next else (res[0], None)


def _temporal_body(hs_ref, wih_ref, whh_ref, bih_ref, bhh_ref,
                   wpo_ref, bpo_ref, wrg_ref, brg_ref, y_ref):
    ht = jnp.zeros((RB, H), _F32)
    for t in range(T):
        xt = hs_ref[t]
        gi = jnp.dot(xt, wih_ref[...], preferred_element_type=_F32) \
            + bih_ref[...]
        gh = jnp.dot(ht, whh_ref[...], preferred_element_type=_F32) \
            + bhh_ref[...]
        r = jax.nn.sigmoid(gi[:, :H] + gh[:, :H])
        z = jax.nn.sigmoid(gi[:, H:2 * H] + gh[:, H:2 * H])
        n = jnp.tanh(gi[:, 2 * H:] + r * gh[:, 2 * H:])
        ht = (1.0 - z) * n + z * ht
    hp = jnp.maximum(
        jnp.dot(ht, wpo_ref[...], preferred_element_type=_F32)
        + bpo_ref[...], 0.0)
    y_ref[...] = jnp.dot(hp, wrg_ref[...], preferred_element_type=_F32) \
        + brg_ref[...]


def _temporal_stage(h_seq, wihT, whhT, bih, bhh, wpoT, bpo, wrgT, brg):
    nblk = NP // RB
    return pl.pallas_call(
        _temporal_body,
        grid=(nblk,),
        in_specs=[
            pl.BlockSpec((T, RB, H), lambda i: (0, i, 0)),
            pl.BlockSpec((H, 3 * H), lambda i: (0, 0)),
            pl.BlockSpec((H, 3 * H), lambda i: (0, 0)),
            pl.BlockSpec((1, 3 * H), lambda i: (0, 0)),
            pl.BlockSpec((1, 3 * H), lambda i: (0, 0)),
            pl.BlockSpec((H, H), lambda i: (0, 0)),
            pl.BlockSpec((1, H), lambda i: (0, 0)),
            pl.BlockSpec((H, 128), lambda i: (0, 0)),
            pl.BlockSpec((1, 128), lambda i: (0, 0)),
        ],
        out_specs=pl.BlockSpec((RB, 128), lambda i: (i, 0)),
        out_shape=jax.ShapeDtypeStruct((NP, 128), _F32),
    )(h_seq, wihT, whhT, bih, bhh, wpoT, bpo, wrgT, brg)


TPE = BASE_CH * CHUNK
XBASE = NW * TPE


def _make_edge_pass(with_deg):
    mesh = plsc.VectorSubcoreMesh(core_axis_name="c", subcore_axis_name="s")
    out_type = [jax.ShapeDtypeStruct((2, NP, H), _F32)]
    if with_deg:
        out_type.append(jax.ShapeDtypeStruct((2, NP), _F32))
    scratch = [
        pltpu.VMEM((TPE,), jnp.int32),
        pltpu.VMEM((TPE,), _F32),
        pltpu.VMEM((CHUNK,), jnp.int32),
        pltpu.VMEM((CHUNK,), jnp.int32),
        pltpu.VMEM((CHUNK,), jnp.int32),
        pltpu.VMEM((CHUNK, H), _F32),
        pltpu.VMEM((CHUNK, H), _F32),
        pltpu.VMEM((CHUNK, H), _F32),
        pltpu.VMEM_SHARED((NP, H), _F32),
    ]
    if with_deg:
        scratch += [
            pltpu.VMEM((CHUNK,), _F32),
            pltpu.VMEM((RPT,), _F32),
            pltpu.VMEM_SHARED((NP,), _F32),
        ]
    scratch += [pltpu.SemaphoreType.DMA] * 9

    def body(u_hbm, src_hbm, dst_hbm, ew_hbm, *rest):
        if with_deg:
            (acc_out, deg_out, src_a, ew_a, d0, d1, d2, r0, r1, r2,
             acc_s, ones_v, zdeg_v, deg_s, *sems) = rest
        else:
            (acc_out, src_a, ew_a, d0, d1, d2, r0, r1, r2,
             acc_s, *sems) = rest
        gsem = sems[0:3]
        dsem = sems[3:6]
        ssem = sems[6:9]
        dstb = (d0, d1, d2)
        rows = (r0, r1, r2)
        cid = lax.axis_index("c")
        sid = lax.axis_index("s")
        wid = sid * _NC + cid
        tb = wid * TPE

        pltpu.sync_copy(src_hbm.at[pl.ds(tb, TPE)], src_a)
        pltpu.sync_copy(ew_hbm.at[pl.ds(tb, TPE)], ew_a)

        def zrow(i, c):
            for j in range(H // 16):
                r0[i, pl.ds(j * 16, 16)] = jnp.zeros((16,), _F32)
            return c
        lax.fori_loop(0, CHUNK, zrow, 0)
        if with_deg:
            def zfill(i, c):
                ones_v[pl.ds(i * 16, 16)] = jnp.ones((16,), _F32)
                return c
            lax.fori_loop(0, CHUNK // 16, zfill, 0)

            def zdeg(i, c):
                zdeg_v[pl.ds(i * 16, 16)] = jnp.zeros((16,), _F32)
                return c
            lax.fori_loop(0, RPT // 16, zdeg, 0)

        base_r = sid * RPT
        for b in range(RPT // CHUNK):
            pltpu.sync_copy(r0, acc_s.at[pl.ds(base_r + b * CHUNK, CHUNK)])
        if with_deg:
            pltpu.sync_copy(zdeg_v, deg_s.at[pl.ds(base_r, RPT)])
        plsc.subcore_barrier()

        def start_gather(k, slot):
            idx = src_a.at[pl.ds(k * CHUNK, CHUNK)]
            pltpu.async_copy(u_hbm.at[idx], rows[slot], gsem[slot])
            pltpu.async_copy(dst_hbm.at[pl.ds(tb + k * CHUNK, CHUNK)],
                             dstb[slot], dsem[slot])

        def wait_gather(slot):
            pltpu.make_async_copy(u_hbm.at[pl.ds(0, CHUNK)], rows[slot],
                                  gsem[slot]).wait()
            pltpu.make_async_copy(dst_hbm.at[pl.ds(0, CHUNK)], dstb[slot],
                                  dsem[slot]).wait()

        def drain_scatter(slot):
            pltpu.make_async_copy(u_hbm.at[pl.ds(0, CHUNK)], rows[slot],
                                  ssem[slot]).wait()

        def scale(k, slot):
            rv = rows[slot]

            def gbody(g, cc):
                ewg = ew_a[pl.ds(k * CHUNK + g * 16, 16)]
                e0 = g * 16
                for kk in range(16):
                    s = ewg[kk]
                    for j in range(H // 16):
                        sl = pl.ds(j * 16, 16)
                        rv[e0 + kk, sl] = rv[e0 + kk, sl] * s
                return cc
            lax.fori_loop(0, CHUNK // 16, gbody, 0)

        start_gather(0, 0)
        start_gather(1, 1)

        def process(j, k):
            wait_gather(j)
            scale(k, j)
            pltpu.async_copy(rows[j], acc_s.at[dstb[j]], ssem[j], add=True)
            if with_deg:
                pltpu.sync_copy(ones_v, deg_s.at[dstb[j]], add=True)
            tgt = (j + 2) % 3

            @pl.when(k == 0)
            def _():
                start_gather(2, 2)

            @pl.when(jnp.logical_and(k >= 1, k + 2 < BASE_CH))
            def _():
                drain_scatter(tgt)
                start_gather(k + 2, tgt)

        def triple(m, c):
            for j in range(3):
                process(j, 3 * m + j)
            return c
        lax.fori_loop(0, BASE_CH // 3, triple, 0)
        for j in range(3):
            drain_scatter(j)

        @pl.when(wid < EXTRA_CH)
        def _():
            bx = XBASE + wid * CHUNK
            pltpu.sync_copy(src_hbm.at[pl.ds(bx, CHUNK)], d1)
            pltpu.sync_copy(dst_hbm.at[pl.ds(bx, CHUNK)], d0)
            pltpu.sync_copy(ew_hbm.at[pl.ds(bx, CHUNK)],
                            ew_a.at[pl.ds(0, CHUNK)])
            pltpu.async_copy(u_hbm.at[d1], r0, gsem[0]).wait()
            scale(0, 0)
            pltpu.sync_copy(r0, acc_s.at[d0], add=True)
            if with_deg:
                pltpu.sync_copy(ones_v, deg_s.at[d0], add=True)

        plsc.subcore_barrier()

        pltpu.sync_copy(acc_s.at[pl.ds(base_r, RPT)],
                        acc_out.at[cid, pl.ds(base_r, RPT)])
        if with_deg:
            pltpu.sync_copy(deg_s.at[pl.ds(base_r, RPT)],
                            deg_out.at[cid, pl.ds(base_r, RPT)])

    return functools.partial(pl.kernel, mesh=mesh, out_type=out_type,
                             scratch_types=scratch)(body)


@functools.lru_cache(maxsize=None)
def _get_edge_pass(with_deg):
    return _make_edge_pass(with_deg)


def _edge_pass(u, src, dst, ew, with_deg):
    if with_deg:
        return _get_edge_pass(True)(u, src, dst, ew)
    (acc,) = _get_edge_pass(False)(u, src, dst, ew)
    return acc, None


def kernel(xs, edge_indices, edge_attrs, W_pre, b_pre, W_e1, b_e1, W_e2,
           b_e2, l1_Wmsg, l1_bmsg, l1_Wih, l1_Whh, l1_bih, l1_bhh,
           l2_Wmsg, l2_bmsg, l2_Wih, l2_Whh, l2_bih, l2_bhh,
           g_Wih, g_Whh, g_bih, g_bhh, W_post, b_post, W_reg, b_reg):
    xs_p = jnp.pad(xs, ((0, 0), (0, NP - N), (0, 0)))
    x_flat = xs_p.reshape(T * NP, D)

    xt_all, u1_all = _pre_stage(
        x_flat, W_pre.T, b_pre[None], l1_Wmsg.T, l1_bmsg[None])
    xt_all = xt_all.reshape(T, NP, H)
    u1_all = u1_all.reshape(T, NP, H)

    ew_all = _ew_stage(
        edge_attrs.reshape(T * E, ED), W_e1.T, b_e1[None], W_e2,
        b_e2.reshape(1, 1)).reshape(T, E)

    l1_wihT, l1_whhT = l1_Wih.T, l1_Whh.T
    l2_wihT, l2_whhT = l2_Wih.T, l2_Whh.T

    outs = []
    for t in range(T):
        src = edge_indices[t, 0]
        dst = edge_indices[t, 1]
        ew = ew_all[t]
        x = xt_all[t]
        u = u1_all[t]

        acc, deg = _edge_pass(u, src, dst, ew, True)
        degs = deg.reshape(2, NP, 1)
        x, u2 = _mpgru_stage(acc, degs, x, l1_wihT, l1_whhT,
                             l1_bih[None], l1_bhh[None],
                             l2_Wmsg.T, l2_bmsg[None])
        acc2, _ = _edge_pass(u2, src, dst, ew, False)
        x, _ = _mpgru_stage(acc2, degs, x, l2_wihT, l2_whhT,
                            l2_bih[None], l2_bhh[None])
        outs.append(x)

    h_seq = jnp.stack(outs, axis=0)
    wrgT = jnp.pad(W_reg.T, ((0, 0), (0, 128 - 3)))
    brg = jnp.pad(b_reg, (0, 128 - 3))[None]
    y = _temporal_stage(h_seq, g_Wih.T, g_Whh.T, g_bih[None], g_bhh[None],
                        W_post.T, b_post[None], wrgT, brg)
    return y[:N, :3]
